# baseline (speedup 1.0000x reference)
import functools

import jax
import jax.numpy as jnp
from jax import lax
from jax.experimental import pallas as pl
from jax.experimental.pallas import tpu as pltpu
from jax.experimental.pallas import tpu_sc as plsc

_B = 16384
_AUDIO = 128
_D = 256
_AD = 32
_NC = 2
_NS = 16
_NW = _NC * _NS
_BPW = _B // _NW
_CH = 128
_NCHUNK = _BPW // _CH


def _sc_gather(item_idx, art_idx, alb_idx, item_tab, art_tab, alb_tab):
    mesh = plsc.VectorSubcoreMesh(core_axis_name="c", subcore_axis_name="s")

    @functools.partial(
        pl.kernel,
        mesh=mesh,
        compiler_params=pltpu.CompilerParams(use_tc_tiling_on_sc=False),
        out_type=(
            jax.ShapeDtypeStruct((_B, _D), jnp.float32),
            jax.ShapeDtypeStruct((_B, _AD), jnp.float32),
            jax.ShapeDtypeStruct((_B, _AD), jnp.float32),
        ),
        scratch_types=[
            pltpu.VMEM((_NCHUNK, _CH), jnp.int32),
            pltpu.VMEM((_NCHUNK, _CH), jnp.int32),
            pltpu.VMEM((_NCHUNK, _CH), jnp.int32),
            pltpu.VMEM((_CH, _D), jnp.float32),
            pltpu.VMEM((_CH, _D), jnp.float32),
            pltpu.VMEM((_BPW, _AD), jnp.float32),
            pltpu.VMEM((_BPW, _AD), jnp.float32),
            pltpu.SemaphoreType.DMA,
            pltpu.SemaphoreType.DMA,
            pltpu.SemaphoreType.DMA,
            pltpu.SemaphoreType.DMA,
            pltpu.SemaphoreType.DMA,
            pltpu.SemaphoreType.DMA,
            pltpu.SemaphoreType.DMA,
            pltpu.SemaphoreType.DMA,
        ],
    )
    def k(item_idx_h, art_idx_h, alb_idx_h, item_tab_h, art_tab_h, alb_tab_h,
          out_item, out_art, out_alb,
          iidx, aidx, bidx, ibuf0, ibuf1, abuf, bbuf,
          g0, g1, o0, o1, sart, salb, soart, soalb):
        wid = lax.axis_index("s") * _NC + lax.axis_index("c")
        base = wid * _BPW
        pltpu.sync_copy(item_idx_h.at[wid], iidx)
        pltpu.sync_copy(art_idx_h.at[wid], aidx)
        pltpu.sync_copy(alb_idx_h.at[wid], bidx)

        art_cps = [
            pltpu.async_copy(art_tab_h.at[aidx.at[c]],
                             abuf.at[pl.ds(c * _CH, _CH)], sart)
            for c in range(_NCHUNK)
        ]
        alb_cps = [
            pltpu.async_copy(alb_tab_h.at[bidx.at[c]],
                             bbuf.at[pl.ds(c * _CH, _CH)], salb)
            for c in range(_NCHUNK)
        ]

        ibufs = (ibuf0, ibuf1)
        gsems = (g0, g1)
        osems = (o0, o1)
        gcp = [None] * _NCHUNK
        ocp = [None] * _NCHUNK
        gcp[0] = pltpu.async_copy(item_tab_h.at[iidx.at[0]], ibufs[0], gsems[0])
        for c in range(_NCHUNK):
            s = c % 2
            if c + 1 < _NCHUNK:
                if c - 1 >= 0:
                    ocp[c - 1].wait()
                gcp[c + 1] = pltpu.async_copy(
                    item_tab_h.at[iidx.at[c + 1]], ibufs[(c + 1) % 2],
                    gsems[(c + 1) % 2])
            gcp[c].wait()
            ocp[c] = pltpu.async_copy(
                ibufs[s], out_item.at[pl.ds(base + c * _CH, _CH)], osems[s])

        for cp in art_cps:
            cp.wait()
        oa = pltpu.async_copy(abuf, out_art.at[pl.ds(base, _BPW)], soart)
        for cp in alb_cps:
            cp.wait()
        ob = pltpu.async_copy(bbuf, out_alb.at[pl.ds(base, _BPW)], soalb)
        ocp[_NCHUNK - 2].wait()
        ocp[_NCHUNK - 1].wait()
        oa.wait()
        ob.wait()

    return k(item_idx, art_idx, alb_idx, item_tab, art_tab, alb_tab)


def _ln(x, g, b):
    m = jnp.mean(x, axis=-1, keepdims=True)
    v = jnp.mean((x - m) ** 2, axis=-1, keepdims=True)
    return (x - m) / jnp.sqrt(v + 1e-5) * g + b


_BM = 1024


def _tc_body(ie, idr, ar, al, wa, w1d, w1i, w1ab, w2, p256, p32, out):
    f32 = jnp.float32
    dense = jnp.dot(ie[...], wa[...], preferred_element_type=f32) + p256[0:1, :]
    dense = jnp.maximum(_ln(dense, p256[1:2, :], p256[2:3, :]), 0.0)
    idv = _ln(idr[...], p256[3:4, :], p256[4:5, :])
    arv = _ln(ar[...], p32[0:1, :], p32[1:2, :])
    alv = _ln(al[...], p32[2:3, :], p32[3:4, :])
    ab = jnp.concatenate([arv, alv], axis=-1)
    h = (jnp.dot(dense, w1d[...], preferred_element_type=f32)
         + jnp.dot(idv, w1i[...], preferred_element_type=f32)
         + jnp.dot(ab, w1ab[...], preferred_element_type=f32)
         + p256[5:6, :])
    h = jnp.maximum(_ln(h, p256[6:7, :], p256[7:8, :]), 0.0)
    h = jnp.dot(h, w2[...], preferred_element_type=f32) + p256[8:9, :]
    h = jnp.maximum(_ln(h, p256[9:10, :], p256[10:11, :]), 0.0)
    n = jnp.sqrt(jnp.sum(h * h, axis=-1, keepdims=True))
    out[...] = h / jnp.maximum(n, 1e-12)


def _tc_forward(item_embed, id_rows, art_rows, alb_rows, Wa, W1d, W1i, W1ab,
                W2, P256, P32):
    grid = (_B // _BM,)
    return pl.pallas_call(
        _tc_body,
        grid=grid,
        in_specs=[
            pl.BlockSpec((_BM, _AUDIO), lambda i: (i, 0)),
            pl.BlockSpec((_BM, _D), lambda i: (i, 0)),
            pl.BlockSpec((_BM, _AD), lambda i: (i, 0)),
            pl.BlockSpec((_BM, _AD), lambda i: (i, 0)),
            pl.BlockSpec((_AUDIO, _D), lambda i: (0, 0)),
            pl.BlockSpec((_D, _D), lambda i: (0, 0)),
            pl.BlockSpec((_D, _D), lambda i: (0, 0)),
            pl.BlockSpec((2 * _AD, _D), lambda i: (0, 0)),
            pl.BlockSpec((_D, _D), lambda i: (0, 0)),
            pl.BlockSpec((16, _D), lambda i: (0, 0)),
            pl.BlockSpec((8, _AD), lambda i: (0, 0)),
        ],
        out_specs=pl.BlockSpec((_BM, _D), lambda i: (i, 0)),
        out_shape=jax.ShapeDtypeStruct((_B, _D), jnp.float32),
    )(item_embed, id_rows, art_rows, alb_rows, Wa, W1d, W1i, W1ab, W2, P256,
      P32)


def kernel(item_embed, target_item_id, item_artist_id, item_album_id,
           item_table, artist_table, album_table, Wa, ba, g_audio, b_audio,
           g_id, b_id, g_art, b_art, g_alb, b_alb, W1, b1, g1, be1, W2, b2,
           g2, be2):
    iidx = target_item_id.astype(jnp.int32).reshape(_NW, _NCHUNK, _CH)
    aidx = item_artist_id.astype(jnp.int32).reshape(_NW, _NCHUNK, _CH)
    bidx = item_album_id.astype(jnp.int32).reshape(_NW, _NCHUNK, _CH)
    id_rows, art_rows, alb_rows = _sc_gather(
        iidx, aidx, bidx, item_table, artist_table, album_table)

    W1d = W1[:_D]
    W1i = W1[_D:2 * _D]
    W1ab = W1[2 * _D:]
    P256 = jnp.concatenate([
        jnp.stack([ba, g_audio, b_audio, g_id, b_id, b1, g1, be1, b2, g2,
                   be2]),
        jnp.zeros((5, _D), jnp.float32),
    ])
    P32 = jnp.concatenate([
        jnp.stack([g_art, b_art, g_alb, b_alb]),
        jnp.zeros((4, _AD), jnp.float32),
    ])
    return _tc_forward(item_embed, id_rows, art_rows, alb_rows, Wa, W1d, W1i,
                       W1ab, W2, P256, P32)


# speedup vs baseline: 12.3965x; 12.3965x over previous
import functools

import jax
import jax.numpy as jnp
from jax import lax
from jax.experimental import pallas as pl
from jax.experimental.pallas import tpu as pltpu
from jax.experimental.pallas import tpu_sc as plsc

_B = 16384
_AUDIO = 128
_D = 256
_AD = 32
_NC = 2
_NS = 16
_NW = _NC * _NS
_BPW = _B // _NW
_CH = 128
_NCHUNK = _BPW // _CH


def _sc_gather_item(item_idx, item_tab):
    mesh = plsc.VectorSubcoreMesh(core_axis_name="c", subcore_axis_name="s")

    @functools.partial(
        pl.kernel,
        mesh=mesh,
        out_type=jax.ShapeDtypeStruct((_B, _D), jnp.float32),
        scratch_types=[
            pltpu.VMEM((_NCHUNK, _CH), jnp.int32),
            pltpu.VMEM((_CH, _D), jnp.float32),
            pltpu.VMEM((_CH, _D), jnp.float32),
            pltpu.SemaphoreType.DMA,
            pltpu.SemaphoreType.DMA,
            pltpu.SemaphoreType.DMA,
            pltpu.SemaphoreType.DMA,
        ],
    )
    def k(item_idx_h, item_tab_h, out_item, iidx, ibuf0, ibuf1, g0, g1, o0,
          o1):
        wid = lax.axis_index("s") * _NC + lax.axis_index("c")
        base = wid * _BPW
        pltpu.sync_copy(item_idx_h.at[wid], iidx)

        ibufs = (ibuf0, ibuf1)
        gsems = (g0, g1)
        osems = (o0, o1)
        gcp = [None] * _NCHUNK
        ocp = [None] * _NCHUNK
        gcp[0] = pltpu.async_copy(item_tab_h.at[iidx.at[0]], ibufs[0], gsems[0])
        for c in range(_NCHUNK):
            s = c % 2
            if c + 1 < _NCHUNK:
                if c - 1 >= 0:
                    ocp[c - 1].wait()
                gcp[c + 1] = pltpu.async_copy(
                    item_tab_h.at[iidx.at[c + 1]], ibufs[(c + 1) % 2],
                    gsems[(c + 1) % 2])
            gcp[c].wait()
            ocp[c] = pltpu.async_copy(
                ibufs[s], out_item.at[pl.ds(base + c * _CH, _CH)], osems[s])

        ocp[_NCHUNK - 2].wait()
        ocp[_NCHUNK - 1].wait()

    return k(item_idx, item_tab)


def _ln(x, g, b):
    m = jnp.mean(x, axis=-1, keepdims=True)
    v = jnp.mean((x - m) ** 2, axis=-1, keepdims=True)
    return (x - m) / jnp.sqrt(v + 1e-5) * g + b


_BM = 1024


def _tc_body(ie, idr, ar, al, wa, w1d, w1i, w1ab, w2, p256, p32, out):
    f32 = jnp.float32
    dense = jnp.dot(ie[...], wa[...], preferred_element_type=f32) + p256[0:1, :]
    dense = jnp.maximum(_ln(dense, p256[1:2, :], p256[2:3, :]), 0.0)
    idv = _ln(idr[...], p256[3:4, :], p256[4:5, :])
    arv = _ln(ar[...], p32[0:1, :], p32[1:2, :])
    alv = _ln(al[...], p32[2:3, :], p32[3:4, :])
    ab = jnp.concatenate([arv, alv], axis=-1)
    h = (jnp.dot(dense, w1d[...], preferred_element_type=f32)
         + jnp.dot(idv, w1i[...], preferred_element_type=f32)
         + jnp.dot(ab, w1ab[...], preferred_element_type=f32)
         + p256[5:6, :])
    h = jnp.maximum(_ln(h, p256[6:7, :], p256[7:8, :]), 0.0)
    h = jnp.dot(h, w2[...], preferred_element_type=f32) + p256[8:9, :]
    h = jnp.maximum(_ln(h, p256[9:10, :], p256[10:11, :]), 0.0)
    n = jnp.sqrt(jnp.sum(h * h, axis=-1, keepdims=True))
    out[...] = h / jnp.maximum(n, 1e-12)


def _tc_forward(item_embed, id_rows, art_rows, alb_rows, Wa, W1d, W1i, W1ab,
                W2, P256, P32):
    grid = (_B // _BM,)
    return pl.pallas_call(
        _tc_body,
        grid=grid,
        in_specs=[
            pl.BlockSpec((_BM, _AUDIO), lambda i: (i, 0)),
            pl.BlockSpec((_BM, _D), lambda i: (i, 0)),
            pl.BlockSpec((_BM, _AD), lambda i: (i, 0)),
            pl.BlockSpec((_BM, _AD), lambda i: (i, 0)),
            pl.BlockSpec((_AUDIO, _D), lambda i: (0, 0)),
            pl.BlockSpec((_D, _D), lambda i: (0, 0)),
            pl.BlockSpec((_D, _D), lambda i: (0, 0)),
            pl.BlockSpec((2 * _AD, _D), lambda i: (0, 0)),
            pl.BlockSpec((_D, _D), lambda i: (0, 0)),
            pl.BlockSpec((16, _D), lambda i: (0, 0)),
            pl.BlockSpec((8, _AD), lambda i: (0, 0)),
        ],
        out_specs=pl.BlockSpec((_BM, _D), lambda i: (i, 0)),
        out_shape=jax.ShapeDtypeStruct((_B, _D), jnp.float32),
    )(item_embed, id_rows, art_rows, alb_rows, Wa, W1d, W1i, W1ab, W2, P256,
      P32)


def kernel(item_embed, target_item_id, item_artist_id, item_album_id,
           item_table, artist_table, album_table, Wa, ba, g_audio, b_audio,
           g_id, b_id, g_art, b_art, g_alb, b_alb, W1, b1, g1, be1, W2, b2,
           g2, be2):
    iidx = target_item_id.astype(jnp.int32).reshape(_NW, _NCHUNK, _CH)
    id_rows = _sc_gather_item(iidx, item_table)
    art_rows = jnp.take(artist_table, item_artist_id, axis=0)
    alb_rows = jnp.take(album_table, item_album_id, axis=0)

    W1d = W1[:_D]
    W1i = W1[_D:2 * _D]
    W1ab = W1[2 * _D:]
    P256 = jnp.concatenate([
        jnp.stack([ba, g_audio, b_audio, g_id, b_id, b1, g1, be1, b2, g2,
                   be2]),
        jnp.zeros((5, _D), jnp.float32),
    ])
    P32 = jnp.concatenate([
        jnp.stack([g_art, b_art, g_alb, b_alb]),
        jnp.zeros((4, _AD), jnp.float32),
    ])
    return _tc_forward(item_embed, id_rows, art_rows, alb_rows, Wa, W1d, W1i,
                       W1ab, W2, P256, P32)
